# baseline (speedup 1.0000x reference)
import functools

import jax
import jax.numpy as jnp
import numpy as np
from jax import lax
from jax.experimental import pallas as pl
from jax.experimental.pallas import tpu as pltpu
from jax.experimental.pallas import tpu_sc as plsc

N = 10000
E = 320000
D = 128
F = 132
FP = 144
NC, NS = 2, 16
NW = NC * NS
NROWP = 10240
STRIDE = NROWP // NS
EPT = 10240
EP = NW * EPT
CH = 64
NCHT = EP // CH
NCH0 = 304
NCH1 = NCHT // NS - NCH0
NB = 8
ECH = EPT // 16
DUMMY = N
INV_SQRT_E = float(1.0 / np.sqrt(E))

_mesh = plsc.VectorSubcoreMesh(
    core_axis_name="c", subcore_axis_name="s", num_cores=NC, num_subcores=NS)
_sc_params = pltpu.CompilerParams(
    needs_layout_passes=False, use_tc_tiling_on_sc=False)


def _pass_a_body(xc_h, yc_h, tc_h, src_h, dst_h,
                 deg_h, ecl_h, mp_h,
                 xc, yc, tc, src_v, dst_v, dacc, eacc,
                 tmpd, tmpe, dstripe, estripe, mp_v, sdeg, secl):
    c = lax.axis_index("c")
    s = lax.axis_index("s")
    wid = c * NS + s
    z16 = jnp.zeros((16,), jnp.float32)
    ones16 = jnp.full((16,), 1.0, jnp.float32)

    pltpu.sync_copy(xc_h, xc)
    pltpu.sync_copy(yc_h, yc)
    pltpu.sync_copy(tc_h, tc)
    pltpu.sync_copy(src_h.at[pl.ds(wid * EPT, EPT)], src_v)
    pltpu.sync_copy(dst_h.at[pl.ds(wid * EPT, EPT)], dst_v)

    def zero_body(i, _):
        dacc[pl.ds(i * 16, 16)] = z16
        eacc[pl.ds(i * 16, 16)] = z16
        return 0
    lax.fori_loop(0, NROWP // 16, zero_body, 0)

    def edge_body(i, _):
        si = src_v[pl.ds(i * 16, 16)]
        di = dst_v[pl.ds(i * 16, 16)]
        xs = plsc.load_gather(xc, [si])
        ys = plsc.load_gather(yc, [si])
        ts = plsc.load_gather(tc, [si])
        xd = plsc.load_gather(xc, [di])
        yd = plsc.load_gather(yc, [di])
        td = plsc.load_gather(tc, [di])
        dx = xs - xd
        dy = ys - yd
        dt = ts - td
        a = dx * dx + dy * dy + dt * dt
        ai = lax.bitcast_convert_type(a, jnp.int32)
        yi = jnp.int32(0x1FBD1DF5) + lax.shift_right_logical(ai, 1)
        y = lax.bitcast_convert_type(yi, jnp.float32)
        y = (y + a / y) * 0.5
        y = (y + a / y) * 0.5
        y = (y + a / y) * 0.5
        y = jnp.where(a > 0.0, y, z16)
        plsc.addupdate_scatter(eacc, [di], y)
        plsc.addupdate_scatter(dacc, [di], ones16)
        return 0
    lax.fori_loop(0, ECH, edge_body, 0)

    pltpu.sync_copy(dacc, sdeg.at[s])
    pltpu.sync_copy(eacc, secl.at[s])
    plsc.subcore_barrier()

    base = s * STRIDE
    for t in range(NS):
        pltpu.sync_copy(sdeg.at[t, pl.ds(base, STRIDE)], tmpd.at[t])
        pltpu.sync_copy(secl.at[t, pl.ds(base, STRIDE)], tmpe.at[t])

    def red_body(i, _):
        accd = z16
        acce = z16
        for t in range(NS):
            accd = accd + tmpd[t, pl.ds(i * 16, 16)]
            acce = acce + tmpe[t, pl.ds(i * 16, 16)]
        dstripe[pl.ds(i * 16, 16)] = accd
        estripe[pl.ds(i * 16, 16)] = acce
        return 0
    lax.fori_loop(0, STRIDE // 16, red_body, 0)

    pltpu.sync_copy(dstripe, deg_h.at[c, pl.ds(base, STRIDE)])
    pltpu.sync_copy(estripe, ecl_h.at[c, pl.ds(base, STRIDE)])

    nvalid = jnp.maximum(0, jnp.minimum(STRIDE, N - base))

    def mean_body(i, carry):
        px, py, pt = carry
        dg = dstripe[pl.ds(i * 16, 16)]
        o = base + i * 16
        px = px + dg * xc[pl.ds(o, 16)]
        py = py + dg * yc[pl.ds(o, 16)]
        pt = pt + dg * tc[pl.ds(o, 16)]
        return px, py, pt
    px, py, pt = lax.fori_loop(0, nvalid // 16, mean_body, (z16, z16, z16))
    mp_v[0, :] = px
    mp_v[1, :] = py
    mp_v[2, :] = pt
    pltpu.sync_copy(mp_v, mp_h.at[wid])


_pass_a = functools.partial(
    pl.kernel,
    out_type=[
        jax.ShapeDtypeStruct((NC, NROWP), jnp.float32),
        jax.ShapeDtypeStruct((NC, NROWP), jnp.float32),
        jax.ShapeDtypeStruct((NW, 3, 16), jnp.float32),
    ],
    mesh=_mesh,
    scratch_types=[
        pltpu.VMEM((N,), jnp.float32),
        pltpu.VMEM((N,), jnp.float32),
        pltpu.VMEM((N,), jnp.float32),
        pltpu.VMEM((EPT,), jnp.int32),
        pltpu.VMEM((EPT,), jnp.int32),
        pltpu.VMEM((NROWP,), jnp.float32),
        pltpu.VMEM((NROWP,), jnp.float32),
        pltpu.VMEM((NS, STRIDE), jnp.float32),
        pltpu.VMEM((NS, STRIDE), jnp.float32),
        pltpu.VMEM((STRIDE,), jnp.float32),
        pltpu.VMEM((STRIDE,), jnp.float32),
        pltpu.VMEM((3, 16), jnp.float32),
        pltpu.VMEM_SHARED((NS, NROWP), jnp.float32),
        pltpu.VMEM_SHARED((NS, NROWP), jnp.float32),
    ],
    compiler_params=_sc_params,
)(_pass_a_body)


def _pass_b_body(xp_ref, w1t_ref, wsm_ref, deg_ref, ecl_ref, mp_ref,
                 q_ref, base_ref):
    xb = xp_ref[...]
    w1t = w1t_ref[...]
    wsm = wsm_ref[...]
    deg = jnp.sum(deg_ref[...], axis=0)
    ecls = jnp.sum(ecl_ref[...], axis=0)
    mm = jnp.sum(mp_ref[...], axis=0)
    mrow = jnp.sum(mm, axis=1, keepdims=True) * (1.0 / E)

    p = jnp.dot(xb, w1t, preferred_element_type=jnp.float32)
    sv0 = jnp.abs(xb[:, 0:1] - mrow[0:1, 0:1]) * INV_SQRT_E
    sv1 = jnp.abs(xb[:, 1:2] - mrow[1:2, 0:1]) * INV_SQRT_E
    sv2 = jnp.abs(xb[:, 2:3] - mrow[2:3, 0:1]) * INV_SQRT_E
    s = sv0 * wsm[0:1, :] + sv1 * wsm[1:2, :] + sv2 * wsm[2:3, :]
    q_ref[...] = s - p
    base_ref[...] = deg * (p + wsm[4:5, :]) + ecls * wsm[3:4, :]


_RB = 1280


def _pass_b(xp, w1t, wsm, deg2, ecl2, mp):
    return pl.pallas_call(
        _pass_b_body,
        grid=(NROWP // _RB,),
        in_specs=[
            pl.BlockSpec((_RB, D), lambda i: (i, 0)),
            pl.BlockSpec((D, FP), lambda i: (0, 0)),
            pl.BlockSpec((8, FP), lambda i: (0, 0)),
            pl.BlockSpec((NC, _RB, 1), lambda i: (0, i, 0)),
            pl.BlockSpec((NC, _RB, 1), lambda i: (0, i, 0)),
            pl.BlockSpec((NW, 3, 16), lambda i: (0, 0, 0)),
        ],
        out_specs=[
            pl.BlockSpec((_RB, FP), lambda i: (i, 0)),
            pl.BlockSpec((_RB, FP), lambda i: (i, 0)),
        ],
        out_shape=[
            jax.ShapeDtypeStruct((NROWP, FP), jnp.float32),
            jax.ShapeDtypeStruct((NROWP, FP), jnp.float32),
        ],
    )(xp, w1t, wsm, deg2, ecl2, mp)


def _pass_c_body(q_h, src_h, dst_h, acc_h, sidx_a, didx_a, sidx_b, didx_b,
                 rows_a, rows_b, semg, semi, acc):
    c = lax.axis_index("c")
    s = lax.axis_index("s")
    z16 = jnp.zeros((16,), jnp.float32)
    rowbase = jnp.where(c == 0, s * NCH0, NS * NCH0 + s * NCH1)
    nblocks = jnp.where(c == 0, NCH0 // NB, NCH1 // NB)

    def idx_fetch(b, si, di):
        pltpu.async_copy(src_h.at[pl.ds(rowbase + b * NB, NB)], si, semi)
        pltpu.async_copy(dst_h.at[pl.ds(rowbase + b * NB, NB)], di, semi)

    def idx_drain(b, si, di):
        pltpu.make_async_copy(src_h.at[pl.ds(rowbase + b * NB, NB)], si,
                              semi).wait()
        pltpu.make_async_copy(dst_h.at[pl.ds(rowbase + b * NB, NB)], di,
                              semi).wait()

    def zrow(i, _):
        for k in range(FP // 16):
            rows_a[i, pl.ds(k * 16, 16)] = z16
        return 0
    lax.fori_loop(0, CH, zrow, 0)
    base = s * STRIDE
    for j in range(STRIDE // CH):
        pltpu.sync_copy(rows_a, acc.at[pl.ds(base + j * CH, CH), :])
    plsc.subcore_barrier()

    idx_fetch(0, sidx_a, didx_a)

    @pl.when(1 < nblocks)
    def _():
        idx_fetch(1, sidx_b, didx_b)

    idx_drain(0, sidx_a, didx_a)
    pltpu.async_copy(q_h.at[sidx_a.at[0]], rows_a, semg)

    def block_pair(bb, _):
        for bi, (sic, dic, sin, din) in enumerate(
                ((sidx_a, didx_a, sidx_b, didx_b),
                 (sidx_b, didx_b, sidx_a, didx_a))):
            b = 2 * bb + bi
            for k in range(NB):
                cur, nxt = (rows_a, rows_b) if k % 2 == 0 else (rows_b, rows_a)
                pltpu.make_async_copy(q_h.at[sic.at[k]], cur, semg).wait()
                if k + 1 < NB:
                    pltpu.async_copy(q_h.at[sic.at[k + 1]], nxt, semg)
                else:
                    @pl.when(b + 1 < nblocks)
                    def _():
                        pltpu.async_copy(q_h.at[sin.at[0]], nxt, semg)
                pltpu.sync_copy(cur, acc.at[dic.at[k]], add=True)

            @pl.when(b + 2 < nblocks)
            def _():
                idx_fetch(b + 2, sic, dic)

            @pl.when(b + 1 < nblocks)
            def _():
                idx_drain(b + 1, sin, din)
        return 0
    lax.fori_loop(0, nblocks // 2, block_pair, 0)

    plsc.subcore_barrier()
    pltpu.sync_copy(acc.at[pl.ds(base, STRIDE), :],
                    acc_h.at[c, pl.ds(base, STRIDE), :])


_pass_c = functools.partial(
    pl.kernel,
    out_type=jax.ShapeDtypeStruct((NC, NROWP, FP), jnp.float32),
    mesh=_mesh,
    scratch_types=[
        pltpu.VMEM((NB, CH), jnp.int32),
        pltpu.VMEM((NB, CH), jnp.int32),
        pltpu.VMEM((NB, CH), jnp.int32),
        pltpu.VMEM((NB, CH), jnp.int32),
        pltpu.VMEM((CH, FP), jnp.float32),
        pltpu.VMEM((CH, FP), jnp.float32),
        pltpu.SemaphoreType.DMA,
        pltpu.SemaphoreType.DMA,
        pltpu.VMEM_SHARED((NROWP, FP), jnp.float32),
    ],
    compiler_params=_sc_params,
)(_pass_c_body)


def _pass_d_body(acc_ref, base_ref, out_ref):
    out_ref[...] = jnp.sum(acc_ref[...], axis=0) + base_ref[...]


def _pass_d(acc, basep):
    return pl.pallas_call(
        _pass_d_body,
        grid=(NROWP // _RB,),
        in_specs=[
            pl.BlockSpec((NC, _RB, FP), lambda i: (0, i, 0)),
            pl.BlockSpec((_RB, FP), lambda i: (i, 0)),
        ],
        out_specs=pl.BlockSpec((_RB, FP), lambda i: (i, 0)),
        out_shape=jax.ShapeDtypeStruct((NROWP, FP), jnp.float32),
    )(acc, basep)


def kernel(x, edge_index, W, b):
    src = edge_index[0].astype(jnp.int32)
    dst = edge_index[1].astype(jnp.int32)
    pad = EP - E
    srcp = jnp.concatenate([src, jnp.zeros((pad,), jnp.int32)])
    dstp = jnp.concatenate([dst, jnp.full((pad,), DUMMY, jnp.int32)])
    src3 = srcp.reshape(NCHT, CH)
    dst3 = dstp.reshape(NCHT, CH)

    xc = x[:, 0]
    yc = x[:, 1]
    tc = x[:, 2]
    xp = jnp.pad(x, ((0, NROWP - N), (0, 0)))

    w1t = jnp.pad(W[:, :D].T, ((0, 0), (0, FP - F)))
    wsm = jnp.zeros((8, FP), jnp.float32)
    wsm = wsm.at[0:3, :F].set(W[:, D:D + 3].T)
    wsm = wsm.at[3, :F].set(W[:, D + 3])
    wsm = wsm.at[4, :F].set(b)

    deg2, ecl2, mp = _pass_a(xc, yc, tc, srcp, dstp)
    qpad, basep = _pass_b(xp, w1t, wsm,
                          deg2.reshape(NC, NROWP, 1),
                          ecl2.reshape(NC, NROWP, 1), mp)
    acc = _pass_c(qpad, src3, dst3)
    outp = _pass_d(acc, basep)
    return outp[:N, :F]


# speedup vs baseline: 1.1245x; 1.1245x over previous
import functools

import jax
import jax.numpy as jnp
import numpy as np
from jax import lax
from jax.experimental import pallas as pl
from jax.experimental.pallas import tpu as pltpu
from jax.experimental.pallas import tpu_sc as plsc

N = 10000
E = 320000
D = 128
F = 132
FP = 144
NC, NS = 2, 16
NW = NC * NS
NROWP = 10240
STRIDE = NROWP // NS
EPT = 10240
EP = NW * EPT
CH = 64
NCHT = EP // CH
NCH0 = 272
NCH1 = NCHT // NS - NCH0
NB = 8
ECH = EPT // 16
DUMMY = N
INV_SQRT_E = float(1.0 / np.sqrt(E))

_mesh = plsc.VectorSubcoreMesh(
    core_axis_name="c", subcore_axis_name="s", num_cores=NC, num_subcores=NS)
_sc_params = pltpu.CompilerParams(
    needs_layout_passes=False, use_tc_tiling_on_sc=False)


def _pass_a_body(xc_h, yc_h, tc_h, src_h, dst_h,
                 deg_h, ecl_h, mp_h,
                 xc, yc, tc, src_v, dst_v, dacc, eacc,
                 tmpd, tmpe, dstripe, estripe, mp_v, sdeg, secl):
    c = lax.axis_index("c")
    s = lax.axis_index("s")
    wid = c * NS + s
    z16 = jnp.zeros((16,), jnp.float32)
    ones16 = jnp.full((16,), 1.0, jnp.float32)

    pltpu.sync_copy(xc_h, xc)
    pltpu.sync_copy(yc_h, yc)
    pltpu.sync_copy(tc_h, tc)
    pltpu.sync_copy(src_h.at[pl.ds(wid * EPT, EPT)], src_v)
    pltpu.sync_copy(dst_h.at[pl.ds(wid * EPT, EPT)], dst_v)

    def zero_body(i, _):
        dacc[pl.ds(i * 16, 16)] = z16
        eacc[pl.ds(i * 16, 16)] = z16
        return 0
    lax.fori_loop(0, NROWP // 16, zero_body, 0)

    def edge_body(i, _):
        si = src_v[pl.ds(i * 16, 16)]
        di = dst_v[pl.ds(i * 16, 16)]
        xs = plsc.load_gather(xc, [si])
        ys = plsc.load_gather(yc, [si])
        ts = plsc.load_gather(tc, [si])
        xd = plsc.load_gather(xc, [di])
        yd = plsc.load_gather(yc, [di])
        td = plsc.load_gather(tc, [di])
        dx = xs - xd
        dy = ys - yd
        dt = ts - td
        a = dx * dx + dy * dy + dt * dt
        ai = lax.bitcast_convert_type(a, jnp.int32)
        yi = jnp.int32(0x1FBD1DF5) + lax.shift_right_logical(ai, 1)
        y = lax.bitcast_convert_type(yi, jnp.float32)
        y = (y + a / y) * 0.5
        y = (y + a / y) * 0.5
        y = (y + a / y) * 0.5
        y = jnp.where(a > 0.0, y, z16)
        plsc.addupdate_scatter(eacc, [di], y)
        plsc.addupdate_scatter(dacc, [di], ones16)
        return 0
    lax.fori_loop(0, ECH, edge_body, 0)

    pltpu.sync_copy(dacc, sdeg.at[s])
    pltpu.sync_copy(eacc, secl.at[s])
    plsc.subcore_barrier()

    base = s * STRIDE
    for t in range(NS):
        pltpu.sync_copy(sdeg.at[t, pl.ds(base, STRIDE)], tmpd.at[t])
        pltpu.sync_copy(secl.at[t, pl.ds(base, STRIDE)], tmpe.at[t])

    def red_body(i, _):
        accd = z16
        acce = z16
        for t in range(NS):
            accd = accd + tmpd[t, pl.ds(i * 16, 16)]
            acce = acce + tmpe[t, pl.ds(i * 16, 16)]
        dstripe[pl.ds(i * 16, 16)] = accd
        estripe[pl.ds(i * 16, 16)] = acce
        return 0
    lax.fori_loop(0, STRIDE // 16, red_body, 0)

    pltpu.sync_copy(dstripe, deg_h.at[c, pl.ds(base, STRIDE)])
    pltpu.sync_copy(estripe, ecl_h.at[c, pl.ds(base, STRIDE)])

    nvalid = jnp.maximum(0, jnp.minimum(STRIDE, N - base))

    def mean_body(i, carry):
        px, py, pt = carry
        dg = dstripe[pl.ds(i * 16, 16)]
        o = base + i * 16
        px = px + dg * xc[pl.ds(o, 16)]
        py = py + dg * yc[pl.ds(o, 16)]
        pt = pt + dg * tc[pl.ds(o, 16)]
        return px, py, pt
    px, py, pt = lax.fori_loop(0, nvalid // 16, mean_body, (z16, z16, z16))
    mp_v[0, :] = px
    mp_v[1, :] = py
    mp_v[2, :] = pt
    pltpu.sync_copy(mp_v, mp_h.at[wid])


_pass_a = functools.partial(
    pl.kernel,
    out_type=[
        jax.ShapeDtypeStruct((NC, NROWP), jnp.float32),
        jax.ShapeDtypeStruct((NC, NROWP), jnp.float32),
        jax.ShapeDtypeStruct((NW, 3, 16), jnp.float32),
    ],
    mesh=_mesh,
    scratch_types=[
        pltpu.VMEM((N,), jnp.float32),
        pltpu.VMEM((N,), jnp.float32),
        pltpu.VMEM((N,), jnp.float32),
        pltpu.VMEM((EPT,), jnp.int32),
        pltpu.VMEM((EPT,), jnp.int32),
        pltpu.VMEM((NROWP,), jnp.float32),
        pltpu.VMEM((NROWP,), jnp.float32),
        pltpu.VMEM((NS, STRIDE), jnp.float32),
        pltpu.VMEM((NS, STRIDE), jnp.float32),
        pltpu.VMEM((STRIDE,), jnp.float32),
        pltpu.VMEM((STRIDE,), jnp.float32),
        pltpu.VMEM((3, 16), jnp.float32),
        pltpu.VMEM_SHARED((NS, NROWP), jnp.float32),
        pltpu.VMEM_SHARED((NS, NROWP), jnp.float32),
    ],
    compiler_params=_sc_params,
)(_pass_a_body)


def _pass_b_body(xp_ref, w1t_ref, wsm_ref, deg_ref, ecl_ref, mp_ref,
                 q_ref, base_ref):
    xb = xp_ref[...]
    w1t = w1t_ref[...]
    wsm = wsm_ref[...]
    deg = jnp.sum(deg_ref[...], axis=0)
    ecls = jnp.sum(ecl_ref[...], axis=0)
    mm = jnp.sum(mp_ref[...], axis=0)
    mrow = jnp.sum(mm, axis=1, keepdims=True) * (1.0 / E)

    p = jnp.dot(xb, w1t, preferred_element_type=jnp.float32)
    sv0 = jnp.abs(xb[:, 0:1] - mrow[0:1, 0:1]) * INV_SQRT_E
    sv1 = jnp.abs(xb[:, 1:2] - mrow[1:2, 0:1]) * INV_SQRT_E
    sv2 = jnp.abs(xb[:, 2:3] - mrow[2:3, 0:1]) * INV_SQRT_E
    s = sv0 * wsm[0:1, :] + sv1 * wsm[1:2, :] + sv2 * wsm[2:3, :]
    q_ref[...] = s - p
    base_ref[...] = deg * (p + wsm[4:5, :]) + ecls * wsm[3:4, :]


_RB = 1280


def _pass_b(xp, w1t, wsm, deg2, ecl2, mp):
    return pl.pallas_call(
        _pass_b_body,
        grid=(NROWP // _RB,),
        in_specs=[
            pl.BlockSpec((_RB, D), lambda i: (i, 0)),
            pl.BlockSpec((D, FP), lambda i: (0, 0)),
            pl.BlockSpec((8, FP), lambda i: (0, 0)),
            pl.BlockSpec((NC, _RB, 1), lambda i: (0, i, 0)),
            pl.BlockSpec((NC, _RB, 1), lambda i: (0, i, 0)),
            pl.BlockSpec((NW, 3, 16), lambda i: (0, 0, 0)),
        ],
        out_specs=[
            pl.BlockSpec((_RB, FP), lambda i: (i, 0)),
            pl.BlockSpec((_RB, FP), lambda i: (i, 0)),
        ],
        out_shape=[
            jax.ShapeDtypeStruct((NROWP, FP), jnp.float32),
            jax.ShapeDtypeStruct((NROWP, FP), jnp.float32),
        ],
    )(xp, w1t, wsm, deg2, ecl2, mp)


def _pass_c_body(q_h, src_h, dst_h, acc_h, sidx_a, didx_a, sidx_b, didx_b,
                 rows_a, rows_b, semg, semi, acc):
    c = lax.axis_index("c")
    s = lax.axis_index("s")
    z16 = jnp.zeros((16,), jnp.float32)
    rowbase = jnp.where(c == 0, s * NCH0, NS * NCH0 + s * NCH1)
    nblocks = jnp.where(c == 0, NCH0 // NB, NCH1 // NB)

    def idx_fetch(b, si, di):
        pltpu.async_copy(src_h.at[pl.ds(rowbase + b * NB, NB)], si, semi)
        pltpu.async_copy(dst_h.at[pl.ds(rowbase + b * NB, NB)], di, semi)

    def idx_drain(b, si, di):
        pltpu.make_async_copy(src_h.at[pl.ds(rowbase + b * NB, NB)], si,
                              semi).wait()
        pltpu.make_async_copy(dst_h.at[pl.ds(rowbase + b * NB, NB)], di,
                              semi).wait()

    def zrow(i, _):
        for k in range(FP // 16):
            rows_a[i, pl.ds(k * 16, 16)] = z16
        return 0
    lax.fori_loop(0, CH, zrow, 0)
    base = s * STRIDE
    for j in range(STRIDE // CH):
        pltpu.sync_copy(rows_a, acc.at[pl.ds(base + j * CH, CH), :])
    plsc.subcore_barrier()

    idx_fetch(0, sidx_a, didx_a)

    @pl.when(1 < nblocks)
    def _():
        idx_fetch(1, sidx_b, didx_b)

    idx_drain(0, sidx_a, didx_a)
    pltpu.async_copy(q_h.at[sidx_a.at[0]], rows_a, semg)

    def block_pair(bb, _):
        for bi, (sic, dic, sin, din) in enumerate(
                ((sidx_a, didx_a, sidx_b, didx_b),
                 (sidx_b, didx_b, sidx_a, didx_a))):
            b = 2 * bb + bi
            for k in range(NB):
                cur, nxt = (rows_a, rows_b) if k % 2 == 0 else (rows_b, rows_a)
                pltpu.make_async_copy(q_h.at[sic.at[k]], cur, semg).wait()
                if k + 1 < NB:
                    pltpu.async_copy(q_h.at[sic.at[k + 1]], nxt, semg)
                else:
                    @pl.when(b + 1 < nblocks)
                    def _():
                        pltpu.async_copy(q_h.at[sin.at[0]], nxt, semg)
                pltpu.sync_copy(cur, acc.at[dic.at[k]], add=True)

            @pl.when(b + 2 < nblocks)
            def _():
                idx_fetch(b + 2, sic, dic)

            @pl.when(b + 1 < nblocks)
            def _():
                idx_drain(b + 1, sin, din)
        return 0
    lax.fori_loop(0, nblocks // 2, block_pair, 0)

    plsc.subcore_barrier()
    pltpu.sync_copy(acc.at[pl.ds(base, STRIDE), :],
                    acc_h.at[c, pl.ds(base, STRIDE), :])


_pass_c = functools.partial(
    pl.kernel,
    out_type=jax.ShapeDtypeStruct((NC, NROWP, FP), jnp.float32),
    mesh=_mesh,
    scratch_types=[
        pltpu.VMEM((NB, CH), jnp.int32),
        pltpu.VMEM((NB, CH), jnp.int32),
        pltpu.VMEM((NB, CH), jnp.int32),
        pltpu.VMEM((NB, CH), jnp.int32),
        pltpu.VMEM((CH, FP), jnp.float32),
        pltpu.VMEM((CH, FP), jnp.float32),
        pltpu.SemaphoreType.DMA,
        pltpu.SemaphoreType.DMA,
        pltpu.VMEM_SHARED((NROWP, FP), jnp.float32),
    ],
    compiler_params=_sc_params,
)(_pass_c_body)


def _pass_d_body(acc_ref, base_ref, out_ref):
    out_ref[...] = jnp.sum(acc_ref[...], axis=0) + base_ref[...]


def _pass_d(acc, basep):
    return pl.pallas_call(
        _pass_d_body,
        grid=(NROWP // _RB,),
        in_specs=[
            pl.BlockSpec((NC, _RB, FP), lambda i: (0, i, 0)),
            pl.BlockSpec((_RB, FP), lambda i: (i, 0)),
        ],
        out_specs=pl.BlockSpec((_RB, FP), lambda i: (i, 0)),
        out_shape=jax.ShapeDtypeStruct((NROWP, FP), jnp.float32),
    )(acc, basep)


def kernel(x, edge_index, W, b):
    src = edge_index[0].astype(jnp.int32)
    dst = edge_index[1].astype(jnp.int32)
    pad = EP - E
    srcp = jnp.concatenate([src, jnp.zeros((pad,), jnp.int32)])
    dstp = jnp.concatenate([dst, jnp.full((pad,), DUMMY, jnp.int32)])
    src3 = srcp.reshape(NCHT, CH)
    dst3 = dstp.reshape(NCHT, CH)

    xc = x[:, 0]
    yc = x[:, 1]
    tc = x[:, 2]
    xp = jnp.pad(x, ((0, NROWP - N), (0, 0)))

    w1t = jnp.pad(W[:, :D].T, ((0, 0), (0, FP - F)))
    wsm = jnp.zeros((8, FP), jnp.float32)
    wsm = wsm.at[0:3, :F].set(W[:, D:D + 3].T)
    wsm = wsm.at[3, :F].set(W[:, D + 3])
    wsm = wsm.at[4, :F].set(b)

    deg2, ecl2, mp = _pass_a(xc, yc, tc, srcp, dstp)
    qpad, basep = _pass_b(xp, w1t, wsm,
                          deg2.reshape(NC, NROWP, 1),
                          ecl2.reshape(NC, NROWP, 1), mp)
    acc = _pass_c(qpad, src3, dst3)
    outp = _pass_d(acc, basep)
    return outp[:N, :F]
